# baseline (speedup 1.0000x reference)
import functools

import jax
import jax.numpy as jnp
from jax import lax
from jax.experimental import pallas as pl
from jax.experimental.pallas import tpu as pltpu
from jax.experimental.pallas import tpu_sc as plsc

F32 = jnp.float32
_C = 80
_NW = 32


def _sc_mesh():
    return plsc.VectorSubcoreMesh(core_axis_name="c", subcore_axis_name="s")


def _wid():
    return lax.axis_index("s") * 2 + lax.axis_index("c")


@functools.lru_cache(maxsize=None)
def _sc_gather2(E, N, W, C=_C):
    nchunks = E // C
    assert nchunks * C == E
    assert W % 16 == 0

    def body(A, B, src, dst, outA,
             si0, di0, bA0, bB0, si1, di1, bA1, bB1, g0, g1):
        w = _wid()
        nk = (nchunks - w + (_NW - 1)) // _NW
        bufs = ((si0, di0, bA0, bB0, g0), (si1, di1, bA1, bB1, g1))

        def fire(k, b):
            si, di, bA, bB, g = bufs[b]
            off = (w + k * _NW) * C
            pltpu.sync_copy(src.at[pl.ds(off, C)], si)
            pltpu.sync_copy(dst.at[pl.ds(off, C)], di)
            pltpu.async_copy(A.at[si], bA, g)
            pltpu.async_copy(B.at[di], bB, g)

        def drain(k, b):
            si, di, bA, bB, g = bufs[b]
            off = (w + k * _NW) * C
            pltpu.make_async_copy(A.at[si], bA, g).wait()
            pltpu.make_async_copy(B.at[di], bB, g).wait()

            def add_row(r, carry):
                for j in range(W // 16):
                    sl = pl.ds(16 * j, 16)
                    bA[r, sl] = bA[r, sl] + bB[r, sl]
                return carry

            lax.fori_loop(0, C, add_row, 0)
            pltpu.sync_copy(bA, outA.at[pl.ds(off, C)])

        @pl.when(nk > 0)
        def _():
            fire(0, 0)

        def pair(i, carry):
            k0 = 2 * i

            @pl.when(k0 + 1 < nk)
            def _():
                fire(k0 + 1, 1)

            drain(k0, 0)

            @pl.when(k0 + 2 < nk)
            def _():
                fire(k0 + 2, 0)

            @pl.when(k0 + 1 < nk)
            def _():
                drain(k0 + 1, 1)

            return carry

        lax.fori_loop(0, (nk + 1) // 2, pair, 0)

    return pl.kernel(
        body,
        out_type=jax.ShapeDtypeStruct((E, W), F32),
        mesh=_sc_mesh(),
        scratch_types=[pltpu.VMEM((C,), jnp.int32),
                       pltpu.VMEM((C,), jnp.int32),
                       pltpu.VMEM((C, W), F32),
                       pltpu.VMEM((C, W), F32),
                       pltpu.VMEM((C,), jnp.int32),
                       pltpu.VMEM((C,), jnp.int32),
                       pltpu.VMEM((C, W), F32),
                       pltpu.VMEM((C, W), F32),
                       pltpu.SemaphoreType.DMA,
                       pltpu.SemaphoreType.DMA],
    )


@functools.lru_cache(maxsize=None)
def _sc_segsum(E, N, W, C=_C):
    nchunks = E // C
    assert nchunks * C == E
    rps = (N // 16) // 8 * 8
    tail = N - 16 * rps

    def body(rows_hbm, idx_hbm, z_hbm, out, di0, r0, di1, r1, l0, l1, acc):
        c = lax.axis_index("c")
        s = lax.axis_index("s")
        w = s * 2 + c
        nk = (nchunks - w + (_NW - 1)) // _NW
        pltpu.sync_copy(z_hbm.at[pl.ds(s * rps, rps)],
                        acc.at[pl.ds(s * rps, rps)])

        @pl.when(s == 0)
        def _():
            pltpu.sync_copy(z_hbm.at[pl.ds(16 * rps, tail)],
                            acc.at[pl.ds(16 * rps, tail)])

        plsc.subcore_barrier()
        bufs = ((di0, r0, l0), (di1, r1, l1))

        def fire(k, b):
            di, rows, l = bufs[b]
            off = (w + k * _NW) * C
            pltpu.async_copy(idx_hbm.at[pl.ds(off, C)], di, l)
            pltpu.async_copy(rows_hbm.at[pl.ds(off, C)], rows, l)

        def process(k, b):
            di, rows, l = bufs[b]
            off = (w + k * _NW) * C
            pltpu.make_async_copy(idx_hbm.at[pl.ds(off, C)], di, l).wait()
            pltpu.make_async_copy(rows_hbm.at[pl.ds(off, C)], rows, l).wait()
            pltpu.sync_copy(rows, acc.at[di], add=True)

        @pl.when(nk > 0)
        def _():
            fire(0, 0)

        def pair(i, carry):
            k0 = 2 * i

            @pl.when(k0 + 1 < nk)
            def _():
                fire(k0 + 1, 1)

            process(k0, 0)

            @pl.when(k0 + 2 < nk)
            def _():
                fire(k0 + 2, 0)

            @pl.when(k0 + 1 < nk)
            def _():
                process(k0 + 1, 1)

            return carry

        lax.fori_loop(0, (nk + 1) // 2, pair, 0)
        plsc.subcore_barrier()
        pltpu.sync_copy(acc.at[pl.ds(s * rps, rps)],
                        out.at[c].at[pl.ds(s * rps, rps)])

        @pl.when(s == 0)
        def _():
            pltpu.sync_copy(acc.at[pl.ds(16 * rps, tail)],
                            out.at[c].at[pl.ds(16 * rps, tail)])

    return pl.kernel(
        body,
        out_type=jax.ShapeDtypeStruct((2, N, W), F32),
        mesh=_sc_mesh(),
        scratch_types=[pltpu.VMEM((C,), jnp.int32),
                       pltpu.VMEM((C, W), F32),
                       pltpu.VMEM((C,), jnp.int32),
                       pltpu.VMEM((C, W), F32),
                       pltpu.SemaphoreType.DMA,
                       pltpu.SemaphoreType.DMA,
                       pltpu.VMEM_SHARED((N, W), F32)],
    )


@functools.lru_cache(maxsize=None)
def _sc_geometry(N, NC, EC, W):
    np_chunks = N // _C
    ne_chunks = EC // _C

    def body(P, cluster, cei0, cei1, z_hbm, pcg, gc0, gc1, idx, buf, sem, acc):
        c = lax.axis_index("c")
        s = lax.axis_index("s")
        w = s * 2 + c

        @pl.when(s == 0)
        def _():
            pltpu.sync_copy(z_hbm, acc)

        plsc.subcore_barrier()
        nk0 = (np_chunks - s + 15) // 16

        def sstep(k, carry):
            off = (s + k * 16) * _C
            pltpu.sync_copy(cluster.at[pl.ds(off, _C)], idx)
            pltpu.sync_copy(P.at[pl.ds(off, _C)], buf)
            pltpu.sync_copy(buf, acc.at[idx], add=True)
            return carry

        lax.fori_loop(0, nk0, sstep, 0)
        plsc.subcore_barrier()

        def gjob(src_ref, nch, out_ref):
            nk = (nch - w + (_NW - 1)) // _NW

            def gstep(k, carry):
                off = (w + k * _NW) * _C
                pltpu.sync_copy(src_ref.at[pl.ds(off, _C)], idx)
                pltpu.async_copy(acc.at[idx], buf, sem).wait()
                pltpu.sync_copy(buf, out_ref.at[pl.ds(off, _C)])
                return carry

            lax.fori_loop(0, nk, gstep, 0)

        gjob(cluster, np_chunks, pcg)
        gjob(cei0, ne_chunks, gc0)
        gjob(cei1, ne_chunks, gc1)

    return pl.kernel(
        body,
        out_type=(jax.ShapeDtypeStruct((N, W), F32),
                  jax.ShapeDtypeStruct((EC, W), F32),
                  jax.ShapeDtypeStruct((EC, W), F32)),
        mesh=_sc_mesh(),
        scratch_types=[pltpu.VMEM((_C,), jnp.int32),
                       pltpu.VMEM((_C, W), F32),
                       pltpu.SemaphoreType.DMA,
                       pltpu.VMEM_SHARED((NC, W), F32)],
    )


@functools.lru_cache(maxsize=None)
def _sc_segsum_small(N, NC, W):
    nchunks = N // _C

    def body(rows_hbm, idx_hbm, z_hbm, out, di, rows, acc):
        c = lax.axis_index("c")
        s = lax.axis_index("s")
        w = s * 2 + c

        @pl.when(s == 0)
        def _():
            pltpu.sync_copy(z_hbm, acc)

        plsc.subcore_barrier()
        nk = (nchunks - w + (_NW - 1)) // _NW

        def step(k, carry):
            off = (w + k * _NW) * _C
            pltpu.sync_copy(idx_hbm.at[pl.ds(off, _C)], di)
            pltpu.sync_copy(rows_hbm.at[pl.ds(off, _C)], rows)
            pltpu.sync_copy(rows, acc.at[di], add=True)
            return carry

        lax.fori_loop(0, nk, step, 0)
        plsc.subcore_barrier()

        @pl.when(s == 0)
        def _():
            pltpu.sync_copy(acc, out.at[c])

    return pl.kernel(
        body,
        out_type=jax.ShapeDtypeStruct((2, NC, W), F32),
        mesh=_sc_mesh(),
        scratch_types=[pltpu.VMEM((_C,), jnp.int32),
                       pltpu.VMEM((_C, W), F32),
                       pltpu.VMEM_SHARED((NC, W), F32)],
    )


def _mm(a, b):
    return jnp.dot(a, b, preferred_element_type=F32)


@functools.lru_cache(maxsize=None)
def _tc_prep(N, W, interpret=False):
    def body(v, wa, wb, a_out, b_out):
        vv = v[...]
        a_out[...] = _mm(vv, wa[...])
        b_out[...] = _mm(vv, wb[...])

    return pl.pallas_call(
        body,
        out_shape=(jax.ShapeDtypeStruct((N, W), F32),
                   jax.ShapeDtypeStruct((N, W), F32)),
        interpret=interpret,
    )


@functools.lru_cache(maxsize=None)
def _tc_edge(E, W, BE, interpret=False):
    grid = E // BE

    def body(e, g, w1, b1, w2, b2, out):
        x = e[...]
        h = jnp.maximum(_mm(x, w1[...]) + g[...] + b1[...], 0.0)
        out[...] = x + _mm(h, w2[...]) + b2[...]

    row = pl.BlockSpec((BE, W), lambda i: (i, 0))
    full = pl.BlockSpec((W, W), lambda i: (0, 0))
    bias = pl.BlockSpec((1, W), lambda i: (0, 0))
    return pl.pallas_call(
        body,
        grid=(grid,),
        in_specs=[row, row, full, bias, full, bias],
        out_specs=row,
        out_shape=jax.ShapeDtypeStruct((E, W), F32),
        interpret=interpret,
    )


@functools.lru_cache(maxsize=None)
def _tc_edge_off(Efull, Eh, W, BE, blk_off, interpret=False):
    grid = Eh // BE

    def body(e, g, w1, b1, w2, b2, out):
        x = e[...]
        h = jnp.maximum(_mm(x, w1[...]) + g[...] + b1[...], 0.0)
        out[...] = x + _mm(h, w2[...]) + b2[...]

    erow = pl.BlockSpec((BE, W), lambda i: (i + blk_off, 0))
    row = pl.BlockSpec((BE, W), lambda i: (i, 0))
    full = pl.BlockSpec((W, W), lambda i: (0, 0))
    bias = pl.BlockSpec((1, W), lambda i: (0, 0))
    return pl.pallas_call(
        body,
        grid=(grid,),
        in_specs=[erow, row, full, bias, full, bias],
        out_specs=row,
        out_shape=jax.ShapeDtypeStruct((Eh, W), F32),
        interpret=interpret,
    )


@functools.lru_cache(maxsize=None)
def _tc_node(N, W, last, interpret=False):
    def body(v, p0, p1, p2, p3, w1a, w1b, b1, w2, b2, wx, wy, fw, b3,
             o_v, o1, o2):
        vv = v[...]
        agg = (p0[...] + p1[...]) + (p2[...] + p3[...])
        h = jnp.maximum(_mm(vv, w1a[...]) + _mm(agg, w1b[...]) + b1[...], 0.0)
        vn = vv + _mm(h, w2[...]) + b2[...]
        o_v[...] = vn
        if last:
            hm = jnp.maximum(_mm(vn, wx[...]) + fw[...], 0.0)
            msg = _mm(hm, wy[...]) + b3[...]
            o1[...] = msg
            o2[...] = msg
        else:
            o1[...] = _mm(vn, wx[...])
            o2[...] = _mm(vn, wy[...])

    return pl.pallas_call(
        body,
        out_shape=(jax.ShapeDtypeStruct((N, W), F32),
                   jax.ShapeDtypeStruct((N, W), F32),
                   jax.ShapeDtypeStruct((N, W), F32)),
        interpret=interpret,
    )


@functools.lru_cache(maxsize=None)
def _tc_geo_fw(N, W, interpret=False):
    def body(pcg, p16, wrow, b1, out):
        g = pcg[...]
        cnt = jnp.maximum(g[:, 2:3], 1.0)
        rx = p16[:, 0:1] - g[:, 0:1] / cnt
        ry = p16[:, 1:2] - g[:, 1:2] / cnt
        feat = jnp.sqrt(rx * rx + ry * ry + 1e-12)
        out[...] = feat * wrow[...] + b1[...]

    return pl.pallas_call(
        body,
        out_shape=jax.ShapeDtypeStruct((N, W), F32),
        interpret=interpret,
    )


@functools.lru_cache(maxsize=None)
def _tc_geo_ec(EC, W, BE, interpret=False):
    grid = EC // BE

    def body(g0, g1, wq1, bq1, wq2, bq2, out):
        a = g0[...]
        b = g1[...]
        ca = jnp.maximum(a[:, 2:3], 1.0)
        cb = jnp.maximum(b[:, 2:3], 1.0)
        dx = a[:, 0:1] / ca - b[:, 0:1] / cb
        dy = a[:, 1:2] / ca - b[:, 1:2] / cb
        fc = jnp.sqrt(dx * dx + dy * dy + 1e-12)
        h = jnp.maximum(fc * wq1[...] + bq1[...], 0.0)
        out[...] = _mm(h, wq2[...]) + bq2[...]

    row = pl.BlockSpec((BE, W), lambda i: (i, 0))
    full = pl.BlockSpec((W, W), lambda i: (0, 0))
    bias = pl.BlockSpec((1, W), lambda i: (0, 0))
    return pl.pallas_call(
        body,
        grid=(grid,),
        in_specs=[row, row, bias, bias, full, bias],
        out_specs=pl.BlockSpec((BE, W), lambda i: (i, 0)),
        out_shape=jax.ShapeDtypeStruct((EC, W), F32),
        interpret=interpret,
    )


@functools.lru_cache(maxsize=None)
def _tc_addbias(NC, W, interpret=False):
    def body(p0, p1, b, out):
        out[...] = p0[...] + p1[...] + b[...]

    return pl.pallas_call(
        body,
        out_shape=jax.ShapeDtypeStruct((NC, W), F32),
        interpret=interpret,
    )


def kernel(v, e, pos, edge_index, cluster, coarse_edge_index,
           We1, be1, We2, be2, Wv1, bv1, Wv2, bv2,
           Wp1, bp1, Wp2, bp2, Wq1, bq1, Wq2, bq2):
    N, W = v.shape
    E = e.shape[0]
    EC = coarse_edge_index.shape[1]
    NC = 2500

    src = edge_index[0].astype(jnp.int32)
    dst = edge_index[1].astype(jnp.int32)
    cl = cluster.astype(jnp.int32)
    c0 = coarse_edge_index[0].astype(jnp.int32)
    c1 = coarse_edge_index[1].astype(jnp.int32)

    z_nw = jnp.zeros((N, W), F32)
    z_ncw = jnp.zeros((NC, W), F32)
    p128 = jnp.concatenate(
        [pos.astype(F32), jnp.ones((N, 1), F32), jnp.zeros((N, W - 3), F32)],
        axis=1)

    pcg, gc0, gc1 = _sc_geometry(N, NC, EC, W)(p128, cl, c0, c1, z_ncw)
    fw = _tc_geo_fw(N, W)(pcg, p128, Wp1[W:W + 1, :], bp1.reshape(1, W))
    e_c = _tc_geo_ec(EC, W, 8000)(gc0, gc1, Wq1, bq1.reshape(1, W),
                                  Wq2, bq2.reshape(1, W))

    a, b = _tc_prep(N, W)(v, We1[0, W:2 * W], We1[0, 2 * W:])
    Eh = E // 2
    nblk_h = Eh // 2000
    src_h = (src[:Eh], src[Eh:])
    dst_h = (dst[:Eh], dst[Eh:])
    e_h = None
    for blk in range(2):
        w1e = We1[blk, :W]
        b1e = be1[blk].reshape(1, W)
        w2e = We2[blk]
        b2e = be2[blk].reshape(1, W)
        g0 = _sc_gather2(Eh, N, W, 128)(a, b, src_h[0], dst_h[0])
        g1 = _sc_gather2(Eh, N, W, 128)(a, b, src_h[1], dst_h[1])
        if e_h is None:
            e_h = (_tc_edge_off(E, Eh, W, 2000, 0)(e, g0, w1e, b1e, w2e, b2e),
                   _tc_edge_off(E, Eh, W, 2000, nblk_h)(e, g1, w1e, b1e,
                                                        w2e, b2e))
        else:
            e_h = (_tc_edge(Eh, W, 2000)(e_h[0], g0, w1e, b1e, w2e, b2e),
                   _tc_edge(Eh, W, 2000)(e_h[1], g1, w1e, b1e, w2e, b2e))
        p0 = _sc_segsum(Eh, N, W, 128)(e_h[0], dst_h[0], z_nw)
        p1 = _sc_segsum(Eh, N, W, 128)(e_h[1], dst_h[1], z_nw)
        last = blk == 1
        if last:
            wx, wy, fb, b3 = Wp1[:W], Wp2, fw, bp2.reshape(1, W)
        else:
            wx, wy = We1[1, W:2 * W], We1[1, 2 * W:]
            fb = jnp.zeros((1, W), F32)
            b3 = fb
        v, a, b = _tc_node(N, W, last)(
            v, p0[0], p0[1], p1[0], p1[1], Wv1[blk, :W], Wv1[blk, W:],
            bv1[blk].reshape(1, W), Wv2[blk], bv2[blk].reshape(1, W),
            wx, wy, fb, b3)
    e = jnp.concatenate(e_h, axis=0)

    vpart = _sc_segsum_small(N, NC, W)(a, cl, z_ncw)
    v_c = _tc_addbias(NC, W)(vpart[0], vpart[1], jnp.zeros((1, W), F32))
    return (v_c, e_c, v, e)


# speedup vs baseline: 1.0199x; 1.0199x over previous
import functools

import jax
import jax.numpy as jnp
from jax import lax
from jax.experimental import pallas as pl
from jax.experimental.pallas import tpu as pltpu
from jax.experimental.pallas import tpu_sc as plsc

F32 = jnp.float32
_C = 80
_NW = 32


def _sc_mesh():
    return plsc.VectorSubcoreMesh(core_axis_name="c", subcore_axis_name="s")


def _wid():
    return lax.axis_index("s") * 2 + lax.axis_index("c")


@functools.lru_cache(maxsize=None)
def _sc_gather2(E, N, W, C=_C):
    nchunks = E // C
    assert nchunks * C == E
    assert W % 16 == 0

    def body(A, B, src, dst, outA,
             si0, di0, bA0, bB0, si1, di1, bA1, bB1, g0, g1):
        w = _wid()
        nk = (nchunks - w + (_NW - 1)) // _NW
        bufs = ((si0, di0, bA0, bB0, g0), (si1, di1, bA1, bB1, g1))

        def fire(k, b):
            si, di, bA, bB, g = bufs[b]
            off = (w + k * _NW) * C
            pltpu.sync_copy(src.at[pl.ds(off, C)], si)
            pltpu.sync_copy(dst.at[pl.ds(off, C)], di)
            pltpu.async_copy(A.at[si], bA, g)
            pltpu.async_copy(B.at[di], bB, g)

        def drain(k, b):
            si, di, bA, bB, g = bufs[b]
            off = (w + k * _NW) * C
            pltpu.make_async_copy(A.at[si], bA, g).wait()
            pltpu.make_async_copy(B.at[di], bB, g).wait()

            def add_row(r, carry):
                for j in range(W // 16):
                    sl = pl.ds(16 * j, 16)
                    bA[r, sl] = bA[r, sl] + bB[r, sl]
                return carry

            lax.fori_loop(0, C, add_row, 0)
            pltpu.sync_copy(bA, outA.at[pl.ds(off, C)])

        @pl.when(nk > 0)
        def _():
            fire(0, 0)

        def pair(i, carry):
            k0 = 2 * i

            @pl.when(k0 + 1 < nk)
            def _():
                fire(k0 + 1, 1)

            drain(k0, 0)

            @pl.when(k0 + 2 < nk)
            def _():
                fire(k0 + 2, 0)

            @pl.when(k0 + 1 < nk)
            def _():
                drain(k0 + 1, 1)

            return carry

        lax.fori_loop(0, (nk + 1) // 2, pair, 0)

    return pl.kernel(
        body,
        out_type=jax.ShapeDtypeStruct((E, W), F32),
        mesh=_sc_mesh(),
        scratch_types=[pltpu.VMEM((C,), jnp.int32),
                       pltpu.VMEM((C,), jnp.int32),
                       pltpu.VMEM((C, W), F32),
                       pltpu.VMEM((C, W), F32),
                       pltpu.VMEM((C,), jnp.int32),
                       pltpu.VMEM((C,), jnp.int32),
                       pltpu.VMEM((C, W), F32),
                       pltpu.VMEM((C, W), F32),
                       pltpu.SemaphoreType.DMA,
                       pltpu.SemaphoreType.DMA],
    )


@functools.lru_cache(maxsize=None)
def _sc_segsum(E, N, W, C=_C):
    nchunks = E // C
    assert nchunks * C == E
    rps = (N // 16) // 8 * 8
    tail = N - 16 * rps

    def body(rows_hbm, idx_hbm, z_hbm, out, di0, r0, di1, r1, l0, l1, acc):
        c = lax.axis_index("c")
        s = lax.axis_index("s")
        w = s * 2 + c
        nk = (nchunks - w + (_NW - 1)) // _NW
        pltpu.sync_copy(z_hbm.at[pl.ds(s * rps, rps)],
                        acc.at[pl.ds(s * rps, rps)])

        @pl.when(s == 0)
        def _():
            pltpu.sync_copy(z_hbm.at[pl.ds(16 * rps, tail)],
                            acc.at[pl.ds(16 * rps, tail)])

        plsc.subcore_barrier()
        bufs = ((di0, r0, l0), (di1, r1, l1))

        def fire(k, b):
            di, rows, l = bufs[b]
            off = (w + k * _NW) * C
            pltpu.async_copy(idx_hbm.at[pl.ds(off, C)], di, l)
            pltpu.async_copy(rows_hbm.at[pl.ds(off, C)], rows, l)

        def process(k, b):
            di, rows, l = bufs[b]
            off = (w + k * _NW) * C
            pltpu.make_async_copy(idx_hbm.at[pl.ds(off, C)], di, l).wait()
            pltpu.make_async_copy(rows_hbm.at[pl.ds(off, C)], rows, l).wait()
            pltpu.sync_copy(rows, acc.at[di], add=True)

        @pl.when(nk > 0)
        def _():
            fire(0, 0)

        def pair(i, carry):
            k0 = 2 * i

            @pl.when(k0 + 1 < nk)
            def _():
                fire(k0 + 1, 1)

            process(k0, 0)

            @pl.when(k0 + 2 < nk)
            def _():
                fire(k0 + 2, 0)

            @pl.when(k0 + 1 < nk)
            def _():
                process(k0 + 1, 1)

            return carry

        lax.fori_loop(0, (nk + 1) // 2, pair, 0)
        plsc.subcore_barrier()
        pltpu.sync_copy(acc.at[pl.ds(s * rps, rps)],
                        out.at[c].at[pl.ds(s * rps, rps)])

        @pl.when(s == 0)
        def _():
            pltpu.sync_copy(acc.at[pl.ds(16 * rps, tail)],
                            out.at[c].at[pl.ds(16 * rps, tail)])

    return pl.kernel(
        body,
        out_type=jax.ShapeDtypeStruct((2, N, W), F32),
        mesh=_sc_mesh(),
        scratch_types=[pltpu.VMEM((C,), jnp.int32),
                       pltpu.VMEM((C, W), F32),
                       pltpu.VMEM((C,), jnp.int32),
                       pltpu.VMEM((C, W), F32),
                       pltpu.SemaphoreType.DMA,
                       pltpu.SemaphoreType.DMA,
                       pltpu.VMEM_SHARED((N, W), F32)],
    )


@functools.lru_cache(maxsize=None)
def _sc_geometry(N, NC, EC, W):
    np_chunks = N // _C
    ne_chunks = EC // _C

    def body(P, cluster, cei0, cei1, z_hbm, pcg, gc0, gc1, idx, buf, sem, acc):
        c = lax.axis_index("c")
        s = lax.axis_index("s")
        w = s * 2 + c

        @pl.when(s == 0)
        def _():
            pltpu.sync_copy(z_hbm, acc)

        plsc.subcore_barrier()
        nk0 = (np_chunks - s + 15) // 16

        def sstep(k, carry):
            off = (s + k * 16) * _C
            pltpu.sync_copy(cluster.at[pl.ds(off, _C)], idx)
            pltpu.sync_copy(P.at[pl.ds(off, _C)], buf)
            pltpu.sync_copy(buf, acc.at[idx], add=True)
            return carry

        lax.fori_loop(0, nk0, sstep, 0)
        plsc.subcore_barrier()

        def gjob(src_ref, nch, out_ref):
            nk = (nch - w + (_NW - 1)) // _NW

            def gstep(k, carry):
                off = (w + k * _NW) * _C
                pltpu.sync_copy(src_ref.at[pl.ds(off, _C)], idx)
                pltpu.async_copy(acc.at[idx], buf, sem).wait()
                pltpu.sync_copy(buf, out_ref.at[pl.ds(off, _C)])
                return carry

            lax.fori_loop(0, nk, gstep, 0)

        gjob(cluster, np_chunks, pcg)
        gjob(cei0, ne_chunks, gc0)
        gjob(cei1, ne_chunks, gc1)

    return pl.kernel(
        body,
        out_type=(jax.ShapeDtypeStruct((N, W), F32),
                  jax.ShapeDtypeStruct((EC, W), F32),
                  jax.ShapeDtypeStruct((EC, W), F32)),
        mesh=_sc_mesh(),
        scratch_types=[pltpu.VMEM((_C,), jnp.int32),
                       pltpu.VMEM((_C, W), F32),
                       pltpu.SemaphoreType.DMA,
                       pltpu.VMEM_SHARED((NC, W), F32)],
    )


@functools.lru_cache(maxsize=None)
def _sc_segsum_small(N, NC, W):
    nchunks = N // _C

    def body(rows_hbm, idx_hbm, z_hbm, out, di, rows, acc):
        c = lax.axis_index("c")
        s = lax.axis_index("s")
        w = s * 2 + c

        @pl.when(s == 0)
        def _():
            pltpu.sync_copy(z_hbm, acc)

        plsc.subcore_barrier()
        nk = (nchunks - w + (_NW - 1)) // _NW

        def step(k, carry):
            off = (w + k * _NW) * _C
            pltpu.sync_copy(idx_hbm.at[pl.ds(off, _C)], di)
            pltpu.sync_copy(rows_hbm.at[pl.ds(off, _C)], rows)
            pltpu.sync_copy(rows, acc.at[di], add=True)
            return carry

        lax.fori_loop(0, nk, step, 0)
        plsc.subcore_barrier()

        @pl.when(s == 0)
        def _():
            pltpu.sync_copy(acc, out.at[c])

    return pl.kernel(
        body,
        out_type=jax.ShapeDtypeStruct((2, NC, W), F32),
        mesh=_sc_mesh(),
        scratch_types=[pltpu.VMEM((_C,), jnp.int32),
                       pltpu.VMEM((_C, W), F32),
                       pltpu.VMEM_SHARED((NC, W), F32)],
    )


def _mm(a, b):
    return jnp.dot(a, b, preferred_element_type=F32)


@functools.lru_cache(maxsize=None)
def _tc_prep(N, W, interpret=False):
    def body(v, wa, wb, a_out, b_out):
        vv = v[...]
        a_out[...] = _mm(vv, wa[...])
        b_out[...] = _mm(vv, wb[...])

    return pl.pallas_call(
        body,
        out_shape=(jax.ShapeDtypeStruct((N, W), F32),
                   jax.ShapeDtypeStruct((N, W), F32)),
        interpret=interpret,
    )


@functools.lru_cache(maxsize=None)
def _tc_edge(E, W, BE, interpret=False):
    grid = E // BE

    def body(e, g, w1, b1, w2, b2, out):
        x = e[...]
        h = jnp.maximum(_mm(x, w1[...]) + g[...] + b1[...], 0.0)
        out[...] = x + _mm(h, w2[...]) + b2[...]

    row = pl.BlockSpec((BE, W), lambda i: (i, 0))
    full = pl.BlockSpec((W, W), lambda i: (0, 0))
    bias = pl.BlockSpec((1, W), lambda i: (0, 0))
    return pl.pallas_call(
        body,
        grid=(grid,),
        in_specs=[row, row, full, bias, full, bias],
        out_specs=row,
        out_shape=jax.ShapeDtypeStruct((E, W), F32),
        interpret=interpret,
    )


@functools.lru_cache(maxsize=None)
def _tc_node(N, W, last, interpret=False):
    def body(v, p0, p1, w1a, w1b, b1, w2, b2, wx, wy, fw, b3, o_v, o1, o2):
        vv = v[...]
        agg = p0[...] + p1[...]
        h = jnp.maximum(_mm(vv, w1a[...]) + _mm(agg, w1b[...]) + b1[...], 0.0)
        vn = vv + _mm(h, w2[...]) + b2[...]
        o_v[...] = vn
        if last:
            hm = jnp.maximum(_mm(vn, wx[...]) + fw[...], 0.0)
            msg = _mm(hm, wy[...]) + b3[...]
            o1[...] = msg
            o2[...] = msg
        else:
            o1[...] = _mm(vn, wx[...])
            o2[...] = _mm(vn, wy[...])

    return pl.pallas_call(
        body,
        out_shape=(jax.ShapeDtypeStruct((N, W), F32),
                   jax.ShapeDtypeStruct((N, W), F32),
                   jax.ShapeDtypeStruct((N, W), F32)),
        interpret=interpret,
    )


@functools.lru_cache(maxsize=None)
def _tc_geo_fw(N, W, interpret=False):
    def body(pcg, p16, wrow, b1, out):
        g = pcg[...]
        cnt = jnp.maximum(g[:, 2:3], 1.0)
        rx = p16[:, 0:1] - g[:, 0:1] / cnt
        ry = p16[:, 1:2] - g[:, 1:2] / cnt
        feat = jnp.sqrt(rx * rx + ry * ry + 1e-12)
        out[...] = feat * wrow[...] + b1[...]

    return pl.pallas_call(
        body,
        out_shape=jax.ShapeDtypeStruct((N, W), F32),
        interpret=interpret,
    )


@functools.lru_cache(maxsize=None)
def _tc_geo_ec(EC, W, BE, interpret=False):
    grid = EC // BE

    def body(g0, g1, wq1, bq1, wq2, bq2, out):
        a = g0[...]
        b = g1[...]
        ca = jnp.maximum(a[:, 2:3], 1.0)
        cb = jnp.maximum(b[:, 2:3], 1.0)
        dx = a[:, 0:1] / ca - b[:, 0:1] / cb
        dy = a[:, 1:2] / ca - b[:, 1:2] / cb
        fc = jnp.sqrt(dx * dx + dy * dy + 1e-12)
        h = jnp.maximum(fc * wq1[...] + bq1[...], 0.0)
        out[...] = _mm(h, wq2[...]) + bq2[...]

    row = pl.BlockSpec((BE, W), lambda i: (i, 0))
    full = pl.BlockSpec((W, W), lambda i: (0, 0))
    bias = pl.BlockSpec((1, W), lambda i: (0, 0))
    return pl.pallas_call(
        body,
        grid=(grid,),
        in_specs=[row, row, bias, bias, full, bias],
        out_specs=pl.BlockSpec((BE, W), lambda i: (i, 0)),
        out_shape=jax.ShapeDtypeStruct((EC, W), F32),
        interpret=interpret,
    )


@functools.lru_cache(maxsize=None)
def _tc_addbias(NC, W, interpret=False):
    def body(p0, p1, b, out):
        out[...] = p0[...] + p1[...] + b[...]

    return pl.pallas_call(
        body,
        out_shape=jax.ShapeDtypeStruct((NC, W), F32),
        interpret=interpret,
    )


def kernel(v, e, pos, edge_index, cluster, coarse_edge_index,
           We1, be1, We2, be2, Wv1, bv1, Wv2, bv2,
           Wp1, bp1, Wp2, bp2, Wq1, bq1, Wq2, bq2):
    N, W = v.shape
    E = e.shape[0]
    EC = coarse_edge_index.shape[1]
    NC = 2500

    src = edge_index[0].astype(jnp.int32)
    dst = edge_index[1].astype(jnp.int32)
    cl = cluster.astype(jnp.int32)
    c0 = coarse_edge_index[0].astype(jnp.int32)
    c1 = coarse_edge_index[1].astype(jnp.int32)

    z_nw = jnp.zeros((N, W), F32)
    z_ncw = jnp.zeros((NC, W), F32)
    p128 = jnp.concatenate(
        [pos.astype(F32), jnp.ones((N, 1), F32), jnp.zeros((N, W - 3), F32)],
        axis=1)

    pcg, gc0, gc1 = _sc_geometry(N, NC, EC, W)(p128, cl, c0, c1, z_ncw)
    fw = _tc_geo_fw(N, W)(pcg, p128, Wp1[W:W + 1, :], bp1.reshape(1, W))
    e_c = _tc_geo_ec(EC, W, 8000)(gc0, gc1, Wq1, bq1.reshape(1, W),
                                  Wq2, bq2.reshape(1, W))

    a, b = _tc_prep(N, W)(v, We1[0, W:2 * W], We1[0, 2 * W:])
    for blk in range(2):
        g = _sc_gather2(E, N, W, 128)(a, b, src, dst)
        e = _tc_edge(E, W, 2560)(e, g, We1[blk, :W],
                                 be1[blk].reshape(1, W), We2[blk],
                                 be2[blk].reshape(1, W))
        part = _sc_segsum(E, N, W, 128)(e, dst, z_nw)
        last = blk == 1
        if last:
            wx, wy, fb, b3 = Wp1[:W], Wp2, fw, bp2.reshape(1, W)
        else:
            wx, wy = We1[1, W:2 * W], We1[1, 2 * W:]
            fb = jnp.zeros((1, W), F32)
            b3 = fb
        v, a, b = _tc_node(N, W, last)(
            v, part[0], part[1], Wv1[blk, :W], Wv1[blk, W:],
            bv1[blk].reshape(1, W), Wv2[blk], bv2[blk].reshape(1, W),
            wx, wy, fb, b3)

    vpart = _sc_segsum_small(N, NC, W)(a, cl, z_ncw)
    v_c = _tc_addbias(NC, W)(vpart[0], vpart[1], jnp.zeros((1, W), F32))
    return (v_c, e_c, v, e)
